# baseline (speedup 1.0000x reference)
import functools

import jax
import jax.numpy as jnp
from jax import lax
from jax.experimental import pallas as pl
from jax.experimental.pallas import tpu as pltpu, tpu_sc as plsc

BATCH = 16384
EMBED_K = 32
NROWS = 1000000
BN = 65536
GRID1 = (NROWS + BN - 1) // BN
PADN = GRID1 * BN
TROW = 128

_info = plsc.get_sparse_core_info()
_NC, _NS, _L = _info.num_cores, _info.num_subcores, _info.num_lanes
_NW = _NC * _NS
_BPW = BATCH // _NW
_CHUNK = 128
_NCHUNK = _BPW // _CHUNK
_GPC = _CHUNK // _L


def _tc_body(w_ref, t_ref, v1_ref, v0_ref):
    r = lax.dot_general(w_ref[...], t_ref[...], (((1,), (0,)), ((), ())),
                        preferred_element_type=jnp.float32)
    v1_ref[...] = r[0]
    v0_ref[...] = r[1]


def _tc_call(w, t_t):
    out1d = jax.ShapeDtypeStruct((PADN,), jnp.float32)
    return pl.pallas_call(
        _tc_body,
        grid=(GRID1,),
        in_specs=[
            pl.BlockSpec((2, EMBED_K), lambda i: (0, 0)),
            pl.BlockSpec((EMBED_K, BN), lambda i: (0, i)),
        ],
        out_specs=[
            pl.BlockSpec((BN,), lambda i: (i,)),
            pl.BlockSpec((BN,), lambda i: (i,)),
        ],
        out_shape=[out1d, out1d],
    )(w, t_t)


def _sc_gather_body(add_partial, idx_hbm, *args):
    if add_partial:
        (v1_hbm, v0_hbm, p1_hbm, p0_hbm, y1_hbm, y0_hbm,
         idx, gidx, b1, b0, p1_v, p0_v, y1_v, y0_v, sem) = args
    else:
        (v1_hbm, v0_hbm, y1_hbm, y0_hbm,
         idx, gidx, b1, b0, p1_v, p0_v, y1_v, y0_v, sem) = args
    wid = lax.axis_index("s") * _NC + lax.axis_index("c")
    base = wid * _BPW

    pltpu.sync_copy(idx_hbm.at[pl.ds(wid * _NCHUNK, _NCHUNK)], idx)
    if add_partial:
        pltpu.sync_copy(p1_hbm.at[pl.ds(base, _BPW)], p1_v)
        pltpu.sync_copy(p0_hbm.at[pl.ds(base, _BPW)], p0_v)

    iota = lax.broadcasted_iota(jnp.int32, (_L,), 0)

    for j in range(_NCHUNK):
        for t in range(_CHUNK // _L):
            sl = pl.ds(t * _L, _L)
            gidx[0, sl] = lax.shift_right_logical(idx[j, sl], 7)
        copies = (
            pltpu.async_copy(v1_hbm.at[gidx.at[0]], b1, sem),
            pltpu.async_copy(v0_hbm.at[gidx.at[0]], b0, sem),
        )
        for c in copies:
            c.wait()

        def group(g, carry, j=j):
            rows = g * _L + iota
            q = idx[j, pl.ds(g * _L, _L)] & (TROW - 1)
            y1 = plsc.load_gather(b1, [rows, q])
            y0 = plsc.load_gather(b0, [rows, q])
            off = (j * _GPC + g) * _L
            if add_partial:
                y1 = y1 + p1_v[pl.ds(off, _L)]
                y0 = y0 + p0_v[pl.ds(off, _L)]
            y1_v[pl.ds(off, _L)] = y1
            y0_v[pl.ds(off, _L)] = y0
            return carry

        lax.fori_loop(0, _GPC, group, 0, unroll=False)

    pltpu.sync_copy(y1_v, y1_hbm.at[pl.ds(base, _BPW)])
    pltpu.sync_copy(y0_v, y0_hbm.at[pl.ds(base, _BPW)])


def _sc_call(idx, v1, v0, partial=None):
    mesh = plsc.VectorSubcoreMesh(core_axis_name="c", subcore_axis_name="s")
    add_partial = partial is not None
    f = functools.partial(
        pl.kernel,
        mesh=mesh,
        compiler_params=pltpu.CompilerParams(needs_layout_passes=False),
        out_type=(
            jax.ShapeDtypeStruct((BATCH,), jnp.float32),
            jax.ShapeDtypeStruct((BATCH,), jnp.float32),
        ),
        scratch_types=[
            pltpu.VMEM((_NCHUNK, _CHUNK), jnp.int32),
            pltpu.VMEM((2, _CHUNK), jnp.int32),
            pltpu.VMEM((_CHUNK, TROW), jnp.float32),
            pltpu.VMEM((_CHUNK, TROW), jnp.float32),
            pltpu.VMEM((_BPW,), jnp.float32),
            pltpu.VMEM((_BPW,), jnp.float32),
            pltpu.VMEM((_BPW,), jnp.float32),
            pltpu.VMEM((_BPW,), jnp.float32),
            pltpu.SemaphoreType.DMA,
        ],
    )(functools.partial(_sc_gather_body, add_partial))
    rb = PADN // TROW
    v1 = v1.reshape(rb, TROW)
    v0 = v0.reshape(rb, TROW)
    if add_partial:
        return f(idx, v1, v0, partial[0], partial[1])
    return f(idx, v1, v0)


@jax.jit
def _run(x, user_table, item_table, W1, W0):
    uidx = x[:, 0].reshape(_NW * _NCHUNK, _CHUNK)
    iidx = x[:, 1].reshape(_NW * _NCHUNK, _CHUNK)
    wu = jnp.concatenate([W1[:, :EMBED_K], W0[:, :EMBED_K]], axis=0)
    wi = jnp.concatenate([W1[:, EMBED_K:], W0[:, EMBED_K:]], axis=0)
    u1, u0 = _tc_call(wu, user_table.T)
    p1, p0 = _sc_call(uidx, u1, u0)
    i1, i0 = _tc_call(wi, item_table.T)
    y1, y0 = _sc_call(iidx, i1, i0, partial=(p1, p0))
    return (y1.reshape(BATCH, 1), y0.reshape(BATCH, 1))


def kernel(x, user_table, item_table, W1, W0):
    return _run(x.astype(jnp.int32), user_table, item_table, W1, W0)


# speedup vs baseline: 1.0240x; 1.0240x over previous
import functools

import jax
import jax.numpy as jnp
from jax import lax
from jax.experimental import pallas as pl
from jax.experimental.pallas import tpu as pltpu, tpu_sc as plsc

BATCH = 16384
EMBED_K = 32
NROWS = 1000000
BN = 32768
GRID1 = (NROWS + BN - 1) // BN
PADN = GRID1 * BN
TROW = 128

_info = plsc.get_sparse_core_info()
_NC, _NS, _L = _info.num_cores, _info.num_subcores, _info.num_lanes
_NW = _NC * _NS
_BPW = BATCH // _NW
_CHUNK = 128
_NCHUNK = _BPW // _CHUNK
_GPC = _CHUNK // _L


def _tc_body(wu_ref, wi_ref, ut_ref, it_ref, u1_ref, u0_ref, i1_ref, i0_ref):
    ru = lax.dot_general(wu_ref[...], ut_ref[...], (((1,), (0,)), ((), ())),
                         preferred_element_type=jnp.float32)
    ri = lax.dot_general(wi_ref[...], it_ref[...], (((1,), (0,)), ((), ())),
                         preferred_element_type=jnp.float32)
    u1_ref[...] = ru[0]
    u0_ref[...] = ru[1]
    i1_ref[...] = ri[0]
    i0_ref[...] = ri[1]


def _tc_call(wu, wi, ut_t, it_t):
    out1d = jax.ShapeDtypeStruct((PADN,), jnp.float32)
    return pl.pallas_call(
        _tc_body,
        grid=(GRID1,),
        in_specs=[
            pl.BlockSpec((2, EMBED_K), lambda i: (0, 0)),
            pl.BlockSpec((2, EMBED_K), lambda i: (0, 0)),
            pl.BlockSpec((EMBED_K, BN), lambda i: (0, i)),
            pl.BlockSpec((EMBED_K, BN), lambda i: (0, i)),
        ],
        out_specs=[
            pl.BlockSpec((BN,), lambda i: (i,)),
            pl.BlockSpec((BN,), lambda i: (i,)),
            pl.BlockSpec((BN,), lambda i: (i,)),
            pl.BlockSpec((BN,), lambda i: (i,)),
        ],
        out_shape=[out1d, out1d, out1d, out1d],
    )(wu, wi, ut_t, it_t)


def _sc_body(uidx_hbm, iidx_hbm, u1_hbm, u0_hbm, i1_hbm, i0_hbm,
             y1_hbm, y0_hbm,
             idx_u, idx_i, gidx, bu1, bu0, bi1, bi0, y1_v, y0_v, sem):
    wid = lax.axis_index("s") * _NC + lax.axis_index("c")
    base = wid * _BPW

    pltpu.sync_copy(uidx_hbm.at[pl.ds(wid * _NCHUNK, _NCHUNK)], idx_u)
    pltpu.sync_copy(iidx_hbm.at[pl.ds(wid * _NCHUNK, _NCHUNK)], idx_i)

    iota = lax.broadcasted_iota(jnp.int32, (_L,), 0)

    for j in range(_NCHUNK):
        for t in range(_CHUNK // _L):
            sl = pl.ds(t * _L, _L)
            gidx[0, sl] = lax.shift_right_logical(idx_u[j, sl], 7)
            gidx[1, sl] = lax.shift_right_logical(idx_i[j, sl], 7)
        copies = (
            pltpu.async_copy(u1_hbm.at[gidx.at[0]], bu1, sem),
            pltpu.async_copy(u0_hbm.at[gidx.at[0]], bu0, sem),
            pltpu.async_copy(i1_hbm.at[gidx.at[1]], bi1, sem),
            pltpu.async_copy(i0_hbm.at[gidx.at[1]], bi0, sem),
        )
        for c in copies:
            c.wait()

        def group(g, carry, j=j):
            rows = g * _L + iota
            qu = idx_u[j, pl.ds(g * _L, _L)] & (TROW - 1)
            qi = idx_i[j, pl.ds(g * _L, _L)] & (TROW - 1)
            y1 = (plsc.load_gather(bu1, [rows, qu])
                  + plsc.load_gather(bi1, [rows, qi]))
            y0 = (plsc.load_gather(bu0, [rows, qu])
                  + plsc.load_gather(bi0, [rows, qi]))
            off = (j * _GPC + g) * _L
            y1_v[pl.ds(off, _L)] = y1
            y0_v[pl.ds(off, _L)] = y0
            return carry

        lax.fori_loop(0, _GPC, group, 0, unroll=False)

    pltpu.sync_copy(y1_v, y1_hbm.at[pl.ds(base, _BPW)])
    pltpu.sync_copy(y0_v, y0_hbm.at[pl.ds(base, _BPW)])


def _sc_call(uidx, iidx, u1, u0, i1, i0):
    mesh = plsc.VectorSubcoreMesh(core_axis_name="c", subcore_axis_name="s")
    f = functools.partial(
        pl.kernel,
        mesh=mesh,
        compiler_params=pltpu.CompilerParams(needs_layout_passes=False),
        out_type=(
            jax.ShapeDtypeStruct((BATCH,), jnp.float32),
            jax.ShapeDtypeStruct((BATCH,), jnp.float32),
        ),
        scratch_types=[
            pltpu.VMEM((_NCHUNK, _CHUNK), jnp.int32),
            pltpu.VMEM((_NCHUNK, _CHUNK), jnp.int32),
            pltpu.VMEM((2, _CHUNK), jnp.int32),
            pltpu.VMEM((_CHUNK, TROW), jnp.float32),
            pltpu.VMEM((_CHUNK, TROW), jnp.float32),
            pltpu.VMEM((_CHUNK, TROW), jnp.float32),
            pltpu.VMEM((_CHUNK, TROW), jnp.float32),
            pltpu.VMEM((_BPW,), jnp.float32),
            pltpu.VMEM((_BPW,), jnp.float32),
            pltpu.SemaphoreType.DMA,
        ],
    )(_sc_body)
    return f(uidx, iidx, u1, u0, i1, i0)


@jax.jit
def _run(x, user_table, item_table, W1, W0):
    uidx = x[:, 0].reshape(_NW * _NCHUNK, _CHUNK)
    iidx = x[:, 1].reshape(_NW * _NCHUNK, _CHUNK)
    wu = jnp.concatenate([W1[:, :EMBED_K], W0[:, :EMBED_K]], axis=0)
    wi = jnp.concatenate([W1[:, EMBED_K:], W0[:, EMBED_K:]], axis=0)
    u1, u0, i1, i0 = _tc_call(wu, wi, user_table.T, item_table.T)
    rb = PADN // TROW
    y1, y0 = _sc_call(uidx, iidx, u1.reshape(rb, TROW), u0.reshape(rb, TROW),
                      i1.reshape(rb, TROW), i0.reshape(rb, TROW))
    return (y1.reshape(BATCH, 1), y0.reshape(BATCH, 1))


def kernel(x, user_table, item_table, W1, W0):
    return _run(x.astype(jnp.int32), user_table, item_table, W1, W0)
